# baseline (speedup 1.0000x reference)
import functools

import jax
import jax.numpy as jnp
from jax import lax
from jax.experimental import pallas as pl
from jax.experimental.pallas import tpu as pltpu
from jax.experimental.pallas import tpu_sc as plsc

EMBED_DIM = 32
NUM_CORES = 2
NUM_SUBCORES = 16
NUM_WORKERS = NUM_CORES * NUM_SUBCORES
LANES = 16
BBLK = 128


def _build(n_batch: int, seq: int):
    n_bt = n_batch // BBLK
    bt_per_w = n_bt // NUM_WORKERS
    units = bt_per_w * seq
    n_groups = units // 4
    assert n_groups % 2 == 0

    mesh = plsc.VectorSubcoreMesh(core_axis_name="c", subcore_axis_name="s")

    @functools.partial(
        pl.kernel,
        mesh=mesh,
        out_type=jax.ShapeDtypeStruct(
            (seq, EMBED_DIM // 8, n_bt, 8, BBLK), jnp.float32
        ),
        scratch_types=[
            pltpu.VMEM((2, 4, BBLK), jnp.int32),
            pltpu.VMEM((2, 4, BBLK, EMBED_DIM), jnp.float32),
            pltpu.VMEM((4, EMBED_DIM, BBLK + 1), jnp.float32),
            pltpu.SemaphoreType.DMA,
            pltpu.SemaphoreType.DMA,
            pltpu.SemaphoreType.DMA,
        ],
        compiler_params=pltpu.CompilerParams(
            use_tc_tiling_on_sc=False, needs_layout_passes=False
        ),
    )
    def emb(idx_hbm, table_hbm, out_hbm, idx_v, rows_v, trans_v, sem_i, sem_g, sem_o):
        wid = lax.axis_index("s") * NUM_CORES + lax.axis_index("c")
        bt0 = wid * bt_per_w

        row_ids = [jnp.arange(LANES, dtype=jnp.int32) + LANES * k for k in range(8)]

        def unit_pos(g, b):
            j = 4 * g + b
            return lax.rem(j, seq), lax.div(j, seq)

        def idx_copy(g, b, p):
            ss, btl = unit_pos(g, b)
            off = ss * n_batch + (bt0 + btl) * BBLK
            return pltpu.make_async_copy(
                idx_hbm.at[pl.ds(off, BBLK)], idx_v.at[p, b], sem_i
            )

        def gather_copy(g, b, p):
            del g
            return pltpu.make_async_copy(
                table_hbm.at[idx_v.at[p, b]], rows_v.at[p, b], sem_g
            )

        def out_copy(g, b, dt):
            ss, btl = unit_pos(g, b)
            return pltpu.make_async_copy(
                trans_v.at[b, pl.ds(dt * 8, 8), pl.ds(0, BBLK)],
                out_hbm.at[ss, dt, bt0 + btl],
                sem_o,
            )

        def transpose_unit(p, b):
            rows = rows_v.at[p, b]
            trans = trans_v.at[b]

            @pl.loop(0, BBLK // 8)
            def _bb(t):
                for u in range(8):
                    bb = 8 * t + u
                    col = jnp.full((LANES,), bb, dtype=jnp.int32)
                    for j in range(2):
                        v = rows[bb, pl.ds(j * LANES, LANES)]
                        plsc.store_scatter(trans, [row_ids[j], col], v)

        for b in range(4):
            idx_copy(0, b, 0).start()
        for b in range(4):
            idx_copy(1, b, 1).start()
        for b in range(4):
            idx_copy(0, b, 0).wait()
        for b in range(4):
            gather_copy(0, b, 0).start()

        @pl.loop(0, n_groups, step=2)
        def _grp(g0):
            for p in range(2):
                g = g0 + p
                for b in range(4):
                    gather_copy(g, b, p).wait()
                @pl.when(g + 1 < n_groups)
                def _():
                    for b in range(4):
                        idx_copy(g + 1, b, 1 - p).wait()
                    for b in range(4):
                        gather_copy(g + 1, b, 1 - p).start()

                @pl.when(g + 2 < n_groups)
                def _():
                    for b in range(4):
                        idx_copy(g + 2, b, p).start()

                for b in range(4):
                    @pl.when(g > 0)
                    def _():
                        for dt in range(4):
                            out_copy(g, b, dt).wait()

                    transpose_unit(p, b)
                    for dt in range(4):
                        out_copy(g, b, dt).start()

        for b in range(4):
            for dt in range(4):
                out_copy(n_groups - 1, b, dt).wait()

    return emb


def kernel(input, table):
    b, s = input.shape
    emb = _build(b, s)
    idx_t = input.T.reshape(s * b).astype(jnp.int32) * 4
    table_pad = jnp.pad(table, ((0, 0), (0, 128 - EMBED_DIM)))
    table4 = table_pad.reshape(4 * table.shape[0], EMBED_DIM)
    out5 = emb(idx_t, table4)
    return out5.transpose(2, 4, 0, 1, 3).reshape(b, s, EMBED_DIM)


# speedup vs baseline: 1.0100x; 1.0100x over previous
import functools

import jax
import jax.numpy as jnp
from jax import lax
from jax.experimental import pallas as pl
from jax.experimental.pallas import tpu as pltpu
from jax.experimental.pallas import tpu_sc as plsc

EMBED_DIM = 32
NUM_CORES = 2
NUM_SUBCORES = 16
NUM_WORKERS = NUM_CORES * NUM_SUBCORES
LANES = 16
BBLK = 128


def _build(n_batch: int, seq: int):
    n_bt = n_batch // BBLK
    bt_per_w = n_bt // NUM_WORKERS
    units = bt_per_w * seq
    n_groups = units // 4
    assert n_groups % 2 == 0

    mesh = plsc.VectorSubcoreMesh(core_axis_name="c", subcore_axis_name="s")

    @functools.partial(
        pl.kernel,
        mesh=mesh,
        out_type=jax.ShapeDtypeStruct(
            (seq, EMBED_DIM // 8, n_bt, 8, BBLK), jnp.float32
        ),
        scratch_types=[
            pltpu.VMEM((2, 4, BBLK), jnp.int32),
            pltpu.VMEM((2, 4, BBLK, EMBED_DIM), jnp.float32),
            pltpu.VMEM((4, EMBED_DIM, BBLK + 1), jnp.float32),
            pltpu.SemaphoreType.DMA,
            pltpu.SemaphoreType.DMA,
            pltpu.SemaphoreType.DMA,
        ],
        compiler_params=pltpu.CompilerParams(
            use_tc_tiling_on_sc=False, needs_layout_passes=False
        ),
    )
    def emb(idx_hbm, table_hbm, out_hbm, idx_v, rows_v, trans_v, sem_i, sem_g, sem_o):
        wid = lax.axis_index("s") * NUM_CORES + lax.axis_index("c")
        bt0 = wid * bt_per_w

        row_ids = [jnp.arange(LANES, dtype=jnp.int32) + LANES * k for k in range(8)]

        def unit_pos(g, b):
            j = 4 * g + b
            return lax.rem(j, seq), lax.div(j, seq)

        def idx_copy(g, b, p):
            ss, btl = unit_pos(g, b)
            off = ss * n_batch + (bt0 + btl) * BBLK
            return pltpu.make_async_copy(
                idx_hbm.at[pl.ds(off, BBLK)], idx_v.at[p, b], sem_i
            )

        def gather_copy(g, b, p):
            del g
            return pltpu.make_async_copy(
                table_hbm.at[idx_v.at[p, b]], rows_v.at[p, b], sem_g
            )

        def out_copy(g, b, dt):
            ss, btl = unit_pos(g, b)
            return pltpu.make_async_copy(
                trans_v.at[b, pl.ds(dt * 8, 8), pl.ds(0, BBLK)],
                out_hbm.at[ss, dt, bt0 + btl],
                sem_o,
            )

        def transpose_unit(p, b):
            rows = rows_v.at[p, b]
            trans = trans_v.at[b]

            @pl.loop(0, BBLK // 4)
            def _bb(t):
                for u in range(4):
                    bb = 4 * t + u
                    col = jnp.full((LANES,), bb, dtype=jnp.int32)
                    for j in range(2):
                        v = rows[bb, pl.ds(j * LANES, LANES)]
                        plsc.store_scatter(trans, [row_ids[j], col], v)

        for b in range(4):
            idx_copy(0, b, 0).start()
        for b in range(4):
            idx_copy(1, b, 1).start()
        for b in range(4):
            idx_copy(0, b, 0).wait()
        for b in range(4):
            gather_copy(0, b, 0).start()

        @pl.loop(0, n_groups, step=2)
        def _grp(g0):
            for p in range(2):
                g = g0 + p
                for b in range(4):
                    gather_copy(g, b, p).wait()
                @pl.when(g + 1 < n_groups)
                def _():
                    for b in range(4):
                        idx_copy(g + 1, b, 1 - p).wait()
                    for b in range(4):
                        gather_copy(g + 1, b, 1 - p).start()

                @pl.when(g + 2 < n_groups)
                def _():
                    for b in range(4):
                        idx_copy(g + 2, b, p).start()

                for b in range(4):
                    @pl.when(g > 0)
                    def _():
                        for dt in range(4):
                            out_copy(g, b, dt).wait()

                    transpose_unit(p, b)
                    for dt in range(4):
                        out_copy(g, b, dt).start()

        for b in range(4):
            for dt in range(4):
                out_copy(n_groups - 1, b, dt).wait()

    return emb


def kernel(input, table):
    b, s = input.shape
    emb = _build(b, s)
    idx_t = input.T.reshape(s * b).astype(jnp.int32) * 4
    table_pad = jnp.pad(table, ((0, 0), (0, 128 - EMBED_DIM)))
    table4 = table_pad.reshape(4 * table.shape[0], EMBED_DIM)
    out5 = emb(idx_t, table4)
    return out5.transpose(2, 4, 0, 1, 3).reshape(b, s, EMBED_DIM)
